# baseline (speedup 1.0000x reference)
import jax
import jax.numpy as jnp
from jax.experimental import pallas as pl
from jax.experimental.pallas import tpu as pltpu

B, H, S, D, Q = 16, 8, 2048, 128, 1


def _body(pos_ref, k_hbm, v_hbm, lk_hbm, lv_hbm, ok_hbm, ov_hbm, csem, rsem):
    copies = []
    for b in range(B):
        copies.append(pltpu.make_async_copy(k_hbm.at[b], ok_hbm.at[b], csem))
        copies.append(pltpu.make_async_copy(v_hbm.at[b], ov_hbm.at[b], csem))
    for c in copies:
        c.start()
    for c in copies:
        c.wait()
    rows = []
    for b in range(B):
        p = pos_ref[b]
        rows.append(pltpu.make_async_copy(
            lk_hbm.at[b], ok_hbm.at[b, :, pl.ds(p, 1), :], rsem))
        rows.append(pltpu.make_async_copy(
            lv_hbm.at[b], ov_hbm.at[b, :, pl.ds(p, 1), :], rsem))
    for r in rows:
        r.start()
    for r in rows:
        r.wait()


def kernel(k_cache, v_cache, latest_k, latest_v, position_ids):
    pos = position_ids.reshape(B).astype(jnp.int32)
    out_shape = [
        jax.ShapeDtypeStruct((B, H, S, D), k_cache.dtype),
        jax.ShapeDtypeStruct((B, H, S, D), v_cache.dtype),
    ]
    k_new, v_new = pl.pallas_call(
        _body,
        grid=(),
        in_specs=[
            pl.BlockSpec(memory_space=pltpu.SMEM),
            pl.BlockSpec(memory_space=pl.ANY),
            pl.BlockSpec(memory_space=pl.ANY),
            pl.BlockSpec(memory_space=pl.ANY),
            pl.BlockSpec(memory_space=pl.ANY),
        ],
        out_specs=[
            pl.BlockSpec(memory_space=pl.ANY),
            pl.BlockSpec(memory_space=pl.ANY),
        ],
        out_shape=out_shape,
        scratch_shapes=[pltpu.SemaphoreType.DMA, pltpu.SemaphoreType.DMA],
    )(pos, k_cache, v_cache, latest_k, latest_v)
    return (k_new, v_new)


# speedup vs baseline: 48.5473x; 48.5473x over previous
import jax
import jax.numpy as jnp
from jax.experimental import pallas as pl
from jax.experimental.pallas import tpu as pltpu

B, H, S, D, Q = 16, 8, 2048, 128, 1
BS = 1024


def _body(pos_ref, k_ref, v_ref, lk_ref, lv_ref, ok_ref, ov_ref):
    b = pl.program_id(0)
    s = pl.program_id(1)
    ok_ref[...] = k_ref[...]
    ov_ref[...] = v_ref[...]
    local = pos_ref[b] - s * BS

    @pl.when((local >= 0) & (local < BS))
    def _():
        ok_ref[0, :, pl.ds(local, 1), :] = lk_ref[0]
        ov_ref[0, :, pl.ds(local, 1), :] = lv_ref[0]


def kernel(k_cache, v_cache, latest_k, latest_v, position_ids):
    pos = position_ids.reshape(B).astype(jnp.int32)
    grid_spec = pltpu.PrefetchScalarGridSpec(
        num_scalar_prefetch=1,
        grid=(B, S // BS),
        in_specs=[
            pl.BlockSpec((1, H, BS, D), lambda b, s, p: (b, 0, s, 0)),
            pl.BlockSpec((1, H, BS, D), lambda b, s, p: (b, 0, s, 0)),
            pl.BlockSpec((1, H, Q, D), lambda b, s, p: (b, 0, 0, 0)),
            pl.BlockSpec((1, H, Q, D), lambda b, s, p: (b, 0, 0, 0)),
        ],
        out_specs=[
            pl.BlockSpec((1, H, BS, D), lambda b, s, p: (b, 0, s, 0)),
            pl.BlockSpec((1, H, BS, D), lambda b, s, p: (b, 0, s, 0)),
        ],
    )
    out_shape = [
        jax.ShapeDtypeStruct((B, H, S, D), k_cache.dtype),
        jax.ShapeDtypeStruct((B, H, S, D), v_cache.dtype),
    ]
    k_new, v_new = pl.pallas_call(
        _body,
        grid_spec=grid_spec,
        out_shape=out_shape,
    )(pos, k_cache, v_cache, latest_k, latest_v)
    return (k_new, v_new)
